# Initial kernel scaffold; baseline (speedup 1.0000x reference)
import jax
import jax.numpy as jnp
from jax.experimental import pallas as pl


def kernel(X, Wq, Wk, Wv, adj):
    raise NotImplementedError("write your pallas kernel here")


# speedup vs baseline: 8.3272x; 8.3272x over previous
import functools
import math

import jax
import jax.numpy as jnp
from jax.experimental import pallas as pl
from jax.experimental.pallas import tpu as pltpu

N = 8192
D = 768
BLK = 128
NB = N // BLK
ROWS1 = 512
INTRA = (1, -1, 2, -2, 0)


def _proj_kernel(x_ref, wq_ref, wk_ref, wv_ref, q_ref, k_ref, v_ref):
    x = x_ref[...]
    dn = (((1,), (1,)), ((), ()))
    q_ref[...] = jax.lax.dot_general(x, wq_ref[...], dn,
                                     preferred_element_type=jnp.float32)
    k_ref[...] = jax.lax.dot_general(x, wk_ref[...], dn,
                                     preferred_element_type=jnp.float32)
    v_ref[...] = jax.lax.dot_general(x, wv_ref[...], dn,
                                     preferred_element_type=jnp.float32)


def _attn_kernel(ids_ref, q_ref, kl_ref, k0, k1, k2, k3,
                 vl_ref, v0, v1, v2, v3, o_ref, *, scale):
    del ids_ref
    q = q_ref[...]
    kl = kl_ref[...]
    scores = []
    for d in INTRA:
        kd = jnp.roll(kl, -d, axis=0) if d else kl
        scores.append(jnp.sum(q * kd, axis=1, keepdims=True))
    for kj in (k0, k1, k2, k3):
        scores.append(jnp.sum(q * kj[...], axis=1, keepdims=True))
    s = jnp.concatenate(scores, axis=1) * scale
    s = s - jnp.max(s, axis=1, keepdims=True)
    e = jnp.exp(s)
    w = e / jnp.sum(e, axis=1, keepdims=True)
    vl = vl_ref[...]
    out = jnp.zeros_like(q)
    for i, d in enumerate(INTRA):
        vd = jnp.roll(vl, -d, axis=0) if d else vl
        out = out + w[:, i:i + 1] * vd
    for j, vj in enumerate((v0, v1, v2, v3)):
        out = out + w[:, 5 + j:6 + j] * vj[...]
    o_ref[...] = out


def kernel(X, Wq, Wk, Wv, adj):
    scale = 1.0 / math.sqrt(D)

    full_w = pl.BlockSpec((D, D), lambda i: (0, 0))
    q, k, v = pl.pallas_call(
        _proj_kernel,
        grid=(N // ROWS1,),
        in_specs=[pl.BlockSpec((ROWS1, D), lambda i: (i, 0)),
                  full_w, full_w, full_w],
        out_specs=[pl.BlockSpec((ROWS1, D), lambda i: (i, 0))] * 3,
        out_shape=[jax.ShapeDtypeStruct((N, D), jnp.float32)] * 3,
    )(X, Wq, Wk, Wv)

    leap_ids = adj[::BLK, 4:8] // BLK

    local = pl.BlockSpec((BLK, D), lambda i, ids: (i, 0))

    def leap_spec(j):
        return pl.BlockSpec((BLK, D), lambda i, ids, j=j: (ids[i, j], 0))

    leaps = [leap_spec(j) for j in range(4)]
    out = pl.pallas_call(
        functools.partial(_attn_kernel, scale=scale),
        grid_spec=pltpu.PrefetchScalarGridSpec(
            num_scalar_prefetch=1,
            grid=(NB,),
            in_specs=[local, local] + leaps + [local] + leaps,
            out_specs=pl.BlockSpec((BLK, D), lambda i, ids: (i, 0)),
        ),
        out_shape=jax.ShapeDtypeStruct((N, D), jnp.float32),
    )(leap_ids, q, k, k, k, k, k, v, v, v, v, v)
    return out


# speedup vs baseline: 10.3208x; 1.2394x over previous
import functools
import math

import jax
import jax.numpy as jnp
from jax.experimental import pallas as pl
from jax.experimental.pallas import tpu as pltpu

N = 8192
D = 768
BLK = 128
NB = N // BLK
ROWS1 = 512
INTRA = (1, -1, 2, -2, 0)
_DN = (((1,), (1,)), ((), ()))


def _proj_kernel(x_ref, wq_ref, wk_ref, wv_ref, q_ref, k_ref, v_ref):
    x = x_ref[...]
    for w_ref, o_ref in ((wq_ref, q_ref), (wk_ref, k_ref), (wv_ref, v_ref)):
        o_ref[...] = jax.lax.dot_general(
            x, w_ref[...], _DN,
            preferred_element_type=jnp.float32).astype(jnp.bfloat16)


def _attn_kernel(ids_ref, q_ref, kl_ref, k0, k1, k2, k3,
                 vl_ref, v0, v1, v2, v3, o_ref, *, scale):
    del ids_ref
    q = q_ref[...]
    s_full = jax.lax.dot_general(q, kl_ref[...], _DN,
                                 preferred_element_type=jnp.float32)
    row = jax.lax.broadcasted_iota(jnp.int32, (BLK, BLK), 0)
    col = jax.lax.broadcasted_iota(jnp.int32, (BLK, BLK), 1)
    masks, scores = [], []
    for d in INTRA:
        m = col == (row + d) % BLK
        masks.append(m)
        scores.append(jnp.sum(jnp.where(m, s_full, 0.0), axis=1,
                              keepdims=True))
    qf = q.astype(jnp.float32)
    for kj in (k0, k1, k2, k3):
        scores.append(jnp.sum(qf * kj[...].astype(jnp.float32), axis=1,
                              keepdims=True))
    s = jnp.concatenate(scores, axis=1) * scale
    s = s - jnp.max(s, axis=1, keepdims=True)
    e = jnp.exp(s)
    w = e / jnp.sum(e, axis=1, keepdims=True)
    wb = jnp.zeros((BLK, BLK), jnp.float32)
    for i, m in enumerate(masks):
        wb = jnp.where(m, w[:, i:i + 1], wb)
    out = jnp.dot(wb, vl_ref[...].astype(jnp.float32),
                  preferred_element_type=jnp.float32)
    for j, vj in enumerate((v0, v1, v2, v3)):
        out = out + w[:, 5 + j:6 + j] * vj[...].astype(jnp.float32)
    o_ref[...] = out


def kernel(X, Wq, Wk, Wv, adj):
    scale = 1.0 / math.sqrt(D)

    full_w = pl.BlockSpec((D, D), lambda i: (0, 0))
    q, k, v = pl.pallas_call(
        _proj_kernel,
        grid=(N // ROWS1,),
        in_specs=[pl.BlockSpec((ROWS1, D), lambda i: (i, 0)),
                  full_w, full_w, full_w],
        out_specs=[pl.BlockSpec((ROWS1, D), lambda i: (i, 0))] * 3,
        out_shape=[jax.ShapeDtypeStruct((N, D), jnp.bfloat16)] * 3,
    )(X, Wq, Wk, Wv)

    leap_ids = adj[::BLK, 4:8] // BLK

    local = pl.BlockSpec((BLK, D), lambda i, ids: (i, 0))

    def leap_spec(j):
        return pl.BlockSpec((BLK, D), lambda i, ids, j=j: (ids[i, j], 0))

    leaps = [leap_spec(j) for j in range(4)]
    out = pl.pallas_call(
        functools.partial(_attn_kernel, scale=scale),
        grid_spec=pltpu.PrefetchScalarGridSpec(
            num_scalar_prefetch=1,
            grid=(NB,),
            in_specs=[local, local] + leaps + [local] + leaps,
            out_specs=pl.BlockSpec((BLK, D), lambda i, ids: (i, 0)),
        ),
        out_shape=jax.ShapeDtypeStruct((N, D), jnp.float32),
    )(leap_ids, q, k, k, k, k, k, v, v, v, v, v)
    return out


# speedup vs baseline: 11.3156x; 1.0964x over previous
import functools
import math

import jax
import jax.numpy as jnp
from jax.experimental import pallas as pl
from jax.experimental.pallas import tpu as pltpu

N = 8192
D = 768
BLK = 128
NB = N // BLK
ROWS1 = 512
QB = 2
INTRA = (1, -1, 2, -2, 0)
_DN = (((1,), (1,)), ((), ()))


def _proj_kernel(x_ref, wq_ref, wk_ref, wv_ref, q_ref, k_ref, v_ref):
    x = x_ref[...].astype(jnp.bfloat16)
    for w_ref, o_ref in ((wq_ref, q_ref), (wk_ref, k_ref), (wv_ref, v_ref)):
        o_ref[...] = jax.lax.dot_general(
            x, w_ref[...], _DN,
            preferred_element_type=jnp.float32).astype(jnp.bfloat16)


def _attn_kernel(ids_ref, q_ref, kl_ref, *args, scale):
    del ids_ref
    kleap = args[:4 * QB]
    vl_ref = args[4 * QB]
    vleap = args[4 * QB + 1:8 * QB + 1]
    o_ref = args[8 * QB + 1]
    row = jax.lax.broadcasted_iota(jnp.int32, (BLK, BLK), 0)
    col = jax.lax.broadcasted_iota(jnp.int32, (BLK, BLK), 1)
    masks = [col == (row + d) % BLK for d in INTRA]
    for jj in range(QB):
        lo = jj * BLK
        q = q_ref[lo:lo + BLK, :]
        kl = kl_ref[lo:lo + BLK, :]
        s_full = jax.lax.dot_general(q, kl, _DN,
                                     preferred_element_type=jnp.float32)
        scores = [jnp.sum(jnp.where(m, s_full, 0.0), axis=1, keepdims=True)
                  for m in masks]
        qf = q.astype(jnp.float32)
        for kj in kleap[4 * jj:4 * jj + 4]:
            scores.append(jnp.sum(qf * kj[...].astype(jnp.float32), axis=1,
                                  keepdims=True))
        s = jnp.concatenate(scores, axis=1) * scale
        s = s - jnp.max(s, axis=1, keepdims=True)
        e = jnp.exp(s)
        w = e / jnp.sum(e, axis=1, keepdims=True)
        wb = jnp.zeros((BLK, BLK), jnp.float32)
        for i, m in enumerate(masks):
            wb = jnp.where(m, w[:, i:i + 1], wb)
        out = jnp.dot(wb, vl_ref[lo:lo + BLK, :].astype(jnp.float32),
                      preferred_element_type=jnp.float32)
        for j, vj in enumerate(vleap[4 * jj:4 * jj + 4]):
            out = out + w[:, 5 + j:6 + j] * vj[...].astype(jnp.float32)
        o_ref[lo:lo + BLK, :] = out


def kernel(X, Wq, Wk, Wv, adj):
    scale = 1.0 / math.sqrt(D)

    full_w = pl.BlockSpec((D, D), lambda i: (0, 0))
    q, k, v = pl.pallas_call(
        _proj_kernel,
        grid=(N // ROWS1,),
        in_specs=[pl.BlockSpec((ROWS1, D), lambda i: (i, 0)),
                  full_w, full_w, full_w],
        out_specs=[pl.BlockSpec((ROWS1, D), lambda i: (i, 0))] * 3,
        out_shape=[jax.ShapeDtypeStruct((N, D), jnp.bfloat16)] * 3,
    )(X, Wq.astype(jnp.bfloat16), Wk.astype(jnp.bfloat16),
      Wv.astype(jnp.bfloat16))

    leap_ids = adj[::BLK, 4:8] // BLK

    local = pl.BlockSpec((QB * BLK, D), lambda i, ids: (i, 0))

    def leap_spec(jj, j):
        return pl.BlockSpec(
            (BLK, D), lambda i, ids, jj=jj, j=j: (ids[i * QB + jj, j], 0))

    leaps = [leap_spec(jj, j) for jj in range(QB) for j in range(4)]
    out = pl.pallas_call(
        functools.partial(_attn_kernel, scale=scale),
        grid_spec=pltpu.PrefetchScalarGridSpec(
            num_scalar_prefetch=1,
            grid=(NB // QB,),
            in_specs=[local, local] + leaps + [local] + leaps,
            out_specs=pl.BlockSpec((QB * BLK, D), lambda i, ids: (i, 0)),
        ),
        out_shape=jax.ShapeDtypeStruct((N, D), jnp.float32),
    )(leap_ids, q, *([k] * (4 * QB + 1)), *([v] * (4 * QB + 1)))
    return out


# speedup vs baseline: 12.6010x; 1.1136x over previous
import functools
import math

import jax
import jax.numpy as jnp
from jax.experimental import pallas as pl
from jax.experimental.pallas import tpu as pltpu

N = 8192
D = 768
BLK = 128
NB = N // BLK
ROWS1 = 1024
QB = 2
INTRA = (1, -1, 2, -2, 0)
_DN = (((1,), (1,)), ((), ()))


def _proj_kernel(x_ref, wq_ref, wk_ref, wv_ref, q_ref, k_ref, v_ref):
    x = x_ref[...].astype(jnp.bfloat16)
    for w_ref, o_ref in ((wq_ref, q_ref), (wk_ref, k_ref), (wv_ref, v_ref)):
        o_ref[...] = jax.lax.dot_general(
            x, w_ref[...], _DN,
            preferred_element_type=jnp.float32).astype(jnp.bfloat16)


def _attn_kernel(ids_ref, q_ref, kl_ref, *args, scale):
    del ids_ref
    kleap = args[:4 * QB]
    vl_ref = args[4 * QB]
    vleap = args[4 * QB + 1:8 * QB + 1]
    o_ref = args[8 * QB + 1]
    row = jax.lax.broadcasted_iota(jnp.int32, (BLK, BLK), 0)
    col = jax.lax.broadcasted_iota(jnp.int32, (BLK, BLK), 1)
    band = jnp.zeros((BLK, BLK), jnp.bool_)
    for d in INTRA:
        band = band | (col == (row + d) % BLK)
    for jj in range(QB):
        lo = jj * BLK
        q = q_ref[lo:lo + BLK, :]
        kl = kl_ref[lo:lo + BLK, :]
        s_full = jax.lax.dot_general(q, kl, _DN,
                                     preferred_element_type=jnp.float32)
        s_band = jnp.where(band, s_full * scale, -1e30)
        m = jnp.max(s_band, axis=1, keepdims=True)
        qf = q.astype(jnp.float32)
        s_leap = [jnp.sum(qf * kj[...].astype(jnp.float32), axis=1,
                          keepdims=True) * scale
                  for kj in kleap[4 * jj:4 * jj + 4]]
        for sj in s_leap:
            m = jnp.maximum(m, sj)
        eb = jnp.exp(s_band - m)
        e_leap = [jnp.exp(sj - m) for sj in s_leap]
        denom = jnp.sum(eb, axis=1, keepdims=True)
        for ej in e_leap:
            denom = denom + ej
        out = jnp.dot(eb, vl_ref[lo:lo + BLK, :].astype(jnp.float32),
                      preferred_element_type=jnp.float32)
        for ej, vj in zip(e_leap, vleap[4 * jj:4 * jj + 4]):
            out = out + ej * vj[...].astype(jnp.float32)
        o_ref[lo:lo + BLK, :] = out / denom


def kernel(X, Wq, Wk, Wv, adj):
    scale = 1.0 / math.sqrt(D)

    full_w = pl.BlockSpec((D, D), lambda i: (0, 0))
    q, k, v = pl.pallas_call(
        _proj_kernel,
        grid=(N // ROWS1,),
        in_specs=[pl.BlockSpec((ROWS1, D), lambda i: (i, 0)),
                  full_w, full_w, full_w],
        out_specs=[pl.BlockSpec((ROWS1, D), lambda i: (i, 0))] * 3,
        out_shape=[jax.ShapeDtypeStruct((N, D), jnp.bfloat16)] * 3,
    )(X, Wq.astype(jnp.bfloat16), Wk.astype(jnp.bfloat16),
      Wv.astype(jnp.bfloat16))

    leap_ids = adj[::BLK, 4:8] // BLK

    local = pl.BlockSpec((QB * BLK, D), lambda i, ids: (i, 0))

    def leap_spec(jj, j):
        return pl.BlockSpec(
            (BLK, D), lambda i, ids, jj=jj, j=j: (ids[i * QB + jj, j], 0))

    leaps = [leap_spec(jj, j) for jj in range(QB) for j in range(4)]
    out = pl.pallas_call(
        functools.partial(_attn_kernel, scale=scale),
        grid_spec=pltpu.PrefetchScalarGridSpec(
            num_scalar_prefetch=1,
            grid=(NB // QB,),
            in_specs=[local, local] + leaps + [local] + leaps,
            out_specs=pl.BlockSpec((QB * BLK, D), lambda i, ids: (i, 0)),
        ),
        out_shape=jax.ShapeDtypeStruct((N, D), jnp.float32),
    )(leap_ids, q, *([k] * (4 * QB + 1)), *([v] * (4 * QB + 1)))
    return out
